# baseline (speedup 1.0000x reference)
import functools

import jax
import jax.numpy as jnp
from jax import lax
from jax.experimental import pallas as pl
from jax.experimental.pallas import tpu as pltpu
from jax.experimental.pallas import tpu_sc as plsc

VOCAB = 128
DIM = 128
SEQ = 512
BATCH = 1024

NC = 2
NS = 16
NW = NC * NS
LANES = 16

ROWS = BATCH * SEQ
RPW = ROWS // NW
CHUNK = 64
NCHUNK = RPW // CHUNK
NBUF = 4
NOUTER = NCHUNK // NBUF


def _emb_body(x_hbm, lw_hbm, pw_hbm, out_hbm,
              idx_v, p_v, b0, b1, b2, b3,
              g0, g1, g2, g3, s0, s1, s2, s3):
    wid = lax.axis_index("s") * NC + lax.axis_index("c")
    row0 = wid * RPW

    bufs = (b0, b1, b2, b3)
    gsem = (g0, g1, g2, g3)
    ssem = (s0, s1, s2, s3)

    pltpu.sync_copy(x_hbm.at[wid], idx_v)
    pltpu.sync_copy(pw_hbm, p_v)

    for c in range(NBUF - 1):
        pltpu.async_copy(lw_hbm.at[idx_v.at[c]], bufs[c], gsem[c])

    def outer(i, carry):
        for b in range(NBUF):
            c = i * NBUF + b
            bn = (b + NBUF - 1) % NBUF
            cn = c + NBUF - 1

            def prefetch():
                pltpu.async_copy(lw_hbm.at[idx_v.at[cn]], bufs[bn], gsem[bn])

            if b == 0:
                @pl.when(i >= 1)
                def _():
                    pltpu.make_async_copy(
                        bufs[bn], out_hbm.at[pl.ds(0, CHUNK)], ssem[bn]
                    ).wait()
                prefetch()
            else:
                @pl.when(i < NOUTER - 1)
                def _():
                    pltpu.make_async_copy(
                        bufs[bn], out_hbm.at[pl.ds(0, CHUNK)], ssem[bn]
                    ).wait()
                    prefetch()

            pltpu.make_async_copy(
                lw_hbm.at[idx_v.at[c]], bufs[b], gsem[b]
            ).wait()
            s_base = (c * CHUNK) % SEQ

            @plsc.parallel_loop(0, CHUNK, 1, unroll=4)
            def add_row(r):
                for k in range(DIM // LANES):
                    sl = pl.ds(k * LANES, LANES)
                    plsc.addupdate(bufs[b].at[r, sl], p_v[s_base + r, sl])

            pltpu.async_copy(
                bufs[b], out_hbm.at[pl.ds(row0 + c * CHUNK, CHUNK)], ssem[b]
            )
        return carry

    lax.fori_loop(0, NOUTER, outer, 0, unroll=False)

    for b in range(NBUF):
        pltpu.make_async_copy(
            bufs[b], out_hbm.at[pl.ds(0, CHUNK)], ssem[b]
        ).wait()


_emb = functools.partial(
    pl.kernel,
    out_type=jax.ShapeDtypeStruct((ROWS, DIM), jnp.float32),
    mesh=plsc.VectorSubcoreMesh(core_axis_name="c", subcore_axis_name="s"),
    scratch_types=[
        pltpu.VMEM((NCHUNK, CHUNK), jnp.int32),
        pltpu.VMEM((SEQ, DIM), jnp.float32),
        pltpu.VMEM((CHUNK, DIM), jnp.float32),
        pltpu.VMEM((CHUNK, DIM), jnp.float32),
        pltpu.VMEM((CHUNK, DIM), jnp.float32),
        pltpu.VMEM((CHUNK, DIM), jnp.float32),
        pltpu.SemaphoreType.DMA,
        pltpu.SemaphoreType.DMA,
        pltpu.SemaphoreType.DMA,
        pltpu.SemaphoreType.DMA,
        pltpu.SemaphoreType.DMA,
        pltpu.SemaphoreType.DMA,
        pltpu.SemaphoreType.DMA,
        pltpu.SemaphoreType.DMA,
    ],
)(_emb_body)


@jax.jit
def kernel(x, embedLettre_w, embedPosition_w):
    xf = x.reshape(NW, NCHUNK, CHUNK)
    out = _emb(xf, embedLettre_w, embedPosition_w)
    return out.reshape(BATCH, SEQ, DIM)


# speedup vs baseline: 2.6625x; 2.6625x over previous
import functools

import jax
import jax.numpy as jnp
from jax import lax
from jax.experimental import pallas as pl
from jax.experimental.pallas import tpu as pltpu
from jax.experimental.pallas import tpu_sc as plsc

VOCAB = 128
DIM = 128
SEQ = 512
BATCH = 1024

NC = 2
NS = 16
NW = NC * NS
LANES = 16

ROWS = BATCH * SEQ
RPW = ROWS // NW
CHUNK = 64
NCHUNK = RPW // CHUNK
NBUF = 4
NOUTER = NCHUNK // NBUF


def _emb_body(x_hbm, lw_hbm, pw_hbm, out_hbm,
              idx_v, p_v, b0, b1, b2, b3,
              g0, g1, g2, g3, s0, s1, s2, s3):
    wid = lax.axis_index("s") * NC + lax.axis_index("c")
    row0 = wid * RPW

    bufs = (b0, b1, b2, b3)
    gsem = (g0, g1, g2, g3)
    ssem = (s0, s1, s2, s3)

    pltpu.sync_copy(x_hbm.at[wid], idx_v)
    pltpu.sync_copy(pw_hbm, p_v)

    lw_t = lw_hbm.at[wid]

    for c in range(NBUF - 1):
        pltpu.async_copy(lw_t.at[idx_v.at[c]], bufs[c], gsem[c])

    def outer(i, carry):
        for b in range(NBUF):
            c = i * NBUF + b
            bn = (b + NBUF - 1) % NBUF
            cn = c + NBUF - 1

            def prefetch():
                pltpu.async_copy(lw_t.at[idx_v.at[cn]], bufs[bn], gsem[bn])

            if b == 0:
                @pl.when(i >= 1)
                def _():
                    pltpu.make_async_copy(
                        bufs[bn], out_hbm.at[pl.ds(0, CHUNK)], ssem[bn]
                    ).wait()
                prefetch()
            else:
                @pl.when(i < NOUTER - 1)
                def _():
                    pltpu.make_async_copy(
                        bufs[bn], out_hbm.at[pl.ds(0, CHUNK)], ssem[bn]
                    ).wait()
                    prefetch()

            pltpu.make_async_copy(
                lw_t.at[idx_v.at[c]], bufs[b], gsem[b]
            ).wait()
            s_base = (c * CHUNK) % SEQ

            @plsc.parallel_loop(0, CHUNK, 1, unroll=4)
            def add_row(r):
                for k in range(DIM // LANES):
                    sl = pl.ds(k * LANES, LANES)
                    plsc.addupdate(bufs[b].at[r, sl], p_v[s_base + r, sl])

            pltpu.async_copy(
                bufs[b], out_hbm.at[pl.ds(row0 + c * CHUNK, CHUNK)], ssem[b]
            )
        return carry

    lax.fori_loop(0, NOUTER, outer, 0, unroll=False)

    for b in range(NBUF):
        pltpu.make_async_copy(
            bufs[b], out_hbm.at[pl.ds(0, CHUNK)], ssem[b]
        ).wait()


_emb = functools.partial(
    pl.kernel,
    out_type=jax.ShapeDtypeStruct((ROWS, DIM), jnp.float32),
    mesh=plsc.VectorSubcoreMesh(core_axis_name="c", subcore_axis_name="s"),
    scratch_types=[
        pltpu.VMEM((NCHUNK, CHUNK), jnp.int32),
        pltpu.VMEM((SEQ, DIM), jnp.float32),
        pltpu.VMEM((CHUNK, DIM), jnp.float32),
        pltpu.VMEM((CHUNK, DIM), jnp.float32),
        pltpu.VMEM((CHUNK, DIM), jnp.float32),
        pltpu.VMEM((CHUNK, DIM), jnp.float32),
        pltpu.SemaphoreType.DMA,
        pltpu.SemaphoreType.DMA,
        pltpu.SemaphoreType.DMA,
        pltpu.SemaphoreType.DMA,
        pltpu.SemaphoreType.DMA,
        pltpu.SemaphoreType.DMA,
        pltpu.SemaphoreType.DMA,
        pltpu.SemaphoreType.DMA,
    ],
)(_emb_body)


@jax.jit
def kernel(x, embedLettre_w, embedPosition_w):
    xf = x.reshape(NW, NCHUNK, CHUNK)
    lwr = jnp.broadcast_to(embedLettre_w[None], (NW, VOCAB, DIM))
    out = _emb(xf, lwr, embedPosition_w)
    return out.reshape(BATCH, SEQ, DIM)
